# speedup vs baseline: 1.4828x; 1.4828x over previous
import functools

import jax
import jax.numpy as jnp
from jax import lax
from jax.experimental import pallas as pl
from jax.experimental.pallas import tpu as pltpu
from jax.experimental.pallas import tpu_sc as plsc

VOCAB = 100000
EMBED_DIM = 768
NUM_TOKENS = 8192

_info = plsc.get_sparse_core_info()
NC, NS = _info.num_cores, _info.num_subcores
NW = NC * NS
ROWS_PER_WORKER = NUM_TOKENS // NW
CHUNK = 64
N_CHUNKS = ROWS_PER_WORKER // CHUNK


def _make_kernel():
    mesh = plsc.VectorSubcoreMesh(core_axis_name="c", subcore_axis_name="s")

    @functools.partial(
        pl.kernel,
        mesh=mesh,
        out_type=jax.ShapeDtypeStruct((NUM_TOKENS, EMBED_DIM), jnp.float32),
        scratch_types=[
            pltpu.VMEM((N_CHUNKS, CHUNK), jnp.int32),
            pltpu.VMEM((2, CHUNK, EMBED_DIM), jnp.float32),
            pltpu.SemaphoreType.DMA,
            pltpu.SemaphoreType.DMA,
        ],
    )
    def emb(ids_hbm, table_hbm, out_hbm, idx_v, rows_v, sem0, sem1):
        wid = lax.axis_index("s") * NC + lax.axis_index("c")
        base = wid * ROWS_PER_WORKER
        pltpu.sync_copy(ids_hbm.at[wid], idx_v)
        sems = (sem0, sem1)
        copies = [None, None]
        copies[0] = pltpu.async_copy(
            table_hbm.at[idx_v.at[0]], rows_v.at[0], sems[0])
        for g in range(N_CHUNKS):
            nxt = (g + 1) % 2
            if g + 1 < N_CHUNKS:
                copies[nxt] = pltpu.async_copy(
                    table_hbm.at[idx_v.at[g + 1]], rows_v.at[nxt], sems[nxt])
            copies[g % 2].wait()
            pltpu.sync_copy(
                rows_v.at[g % 2], out_hbm.at[pl.ds(base + g * CHUNK, CHUNK)])

    return emb


_emb = _make_kernel()


def kernel(token_ids, table):
    batch, seq_len = token_ids.shape
    ids = token_ids.astype(jnp.int32).reshape(NW, N_CHUNKS, CHUNK)
    out = _emb(ids, table)
    return out.reshape(batch, seq_len, EMBED_DIM)
